# Initial kernel scaffold; baseline (speedup 1.0000x reference)
import jax
import jax.numpy as jnp
from jax.experimental import pallas as pl


def kernel(CA, CB, coeff, cutoffs, mask):
    raise NotImplementedError("write your pallas kernel here")


# speedup vs baseline: 1.6224x; 1.6224x over previous
import functools

import numpy as np
import jax
import jax.numpy as jnp
from jax import lax
from jax.experimental import pallas as pl
from jax.experimental.pallas import tpu as pltpu
from jax.experimental.pallas import tpu_sc as plsc

NC = 2
NS = 16
LN = 16
NW = NC * NS
CHUNK = 128
SUB = CHUNK // LN
EPS = np.float32(1e-6)

_PIO2_HI = np.float32(1.5707962513e+00)
_PIO2_LO = np.float32(7.5497894159e-08)
_PS0 = np.float32(1.6666586697e-01)
_PS1 = np.float32(-4.2743422091e-02)
_PS2 = np.float32(-8.6563630030e-03)
_QS1 = np.float32(-7.0662963390e-01)
_MAGIC = np.int32(0x5F3759DF)


def _sqrtf(z):
    i = lax.bitcast_convert_type(z, jnp.int32)
    y = lax.bitcast_convert_type(_MAGIC - lax.shift_right_logical(i, 1),
                                 jnp.float32)
    hz = jnp.float32(0.5) * z
    y = y * (jnp.float32(1.5) - hz * y * y)
    y = y * (jnp.float32(1.5) - hz * y * y)
    s = z * y
    return jnp.float32(0.5) * (s + z / s)


def _acosf(x):
    ax = jnp.abs(x)
    half = jnp.float32(0.5)
    one = jnp.float32(1.0)
    lo = ax < half
    neg = x < jnp.float32(0.0)
    z = jnp.where(lo, x * x, half * jnp.where(neg, one + x, one - x))
    p = z * (_PS0 + z * (_PS1 + z * _PS2))
    q = one + z * _QS1
    r = p / q
    s = _sqrtf(z)
    r1 = _PIO2_HI - (x - (_PIO2_LO - x * r))
    r2 = jnp.float32(2.0) * (_PIO2_HI - (s + (r * s - _PIO2_LO)))
    df = lax.bitcast_convert_type(
        lax.bitcast_convert_type(s, jnp.int32) & jnp.int32(~0xFFF),
        jnp.float32)
    c = (z - df * df) / (s + df)
    r3 = jnp.float32(2.0) * (df + (r * s + c))
    return jnp.where(lo, r1, jnp.where(neg, r2, r3))


def _make_sc_call(Bn, L, nseg, ncut):
    slab = (L // NW) * L
    shift = L.bit_length() - 1

    mesh = plsc.VectorSubcoreMesh(core_axis_name="c", subcore_axis_name="s",
                                  num_cores=NC, num_subcores=NS)

    @functools.partial(
        pl.kernel,
        out_type=jax.ShapeDtypeStruct((NW, LN), jnp.float32),
        mesh=mesh,
        compiler_params=pltpu.CompilerParams(needs_layout_passes=False,
                                             use_tc_tiling_on_sc=False),
        scratch_types=[
            pltpu.VMEM((slab,), jnp.int32),
            pltpu.VMEM((slab + LN,), jnp.int32),
            pltpu.VMEM((6 * Bn, L), jnp.float32),
            pltpu.VMEM((ncut, LN), jnp.float32),
            pltpu.VMEM((CHUNK,), jnp.int32),
            pltpu.VMEM((CHUNK,), jnp.int32),
            pltpu.VMEM((CHUNK,), jnp.float32),
            pltpu.VMEM((CHUNK,), jnp.float32),
            pltpu.VMEM((CHUNK, 8), jnp.float32),
            pltpu.VMEM((LN,), jnp.float32),
            pltpu.SemaphoreType.DMA,
        ],
    )
    def sc_call(mask_hbm, geom_hbm, cut_hbm, coeff_hbm, out_hbm,
                mslab, pairs, geomv, cutv, offs, subs, xls, wts, rows, accv,
                sem):
        wid = lax.axis_index("s") * NC + lax.axis_index("c")
        base_pair = wid * slab

        pltpu.sync_copy(mask_hbm.at[pl.ds(base_pair, slab)], mslab)
        pltpu.sync_copy(geom_hbm, geomv)
        pltpu.sync_copy(cut_hbm, cutv)
        accv[...] = jnp.zeros((LN,), jnp.float32)

        iota = lax.iota(jnp.int32, LN)

        def compact_body(q, cnt):
            v = mslab[pl.ds(q * LN, LN)]
            m = v != 0
            ids = (base_pair + q * LN) + iota
            mi = jnp.where(m, jnp.int32(1), jnp.int32(0))
            incl = plsc.cumsum(mi)
            pos = cnt + incl - jnp.int32(1)
            plsc.store_scatter(pairs.at[pl.ds(0, slab + LN)], [pos], ids,
                               mask=m)
            return cnt + jnp.sum(mi)

        cnt = lax.fori_loop(0, slab // LN, compact_body, jnp.int32(0))

        nch = lax.shift_right_logical(cnt + jnp.int32(CHUNK - 1), 7)

        def make_chunk_body(b):
            rb = 6 * b

            def chunk_body(g, carry):
                cbase = g * CHUNK
                for t in range(SUB):
                    k0 = cbase + t * LN
                    vm = (k0 + iota) < cnt
                    p = jnp.where(vm, pairs[pl.ds(k0, LN)], 0)
                    i = lax.shift_right_logical(p, shift)
                    j = p & jnp.int32(L - 1)

                    def ld(row, idxv):
                        return plsc.load_gather(
                            geomv, [jnp.full((LN,), row, jnp.int32), idxv])

                    cbxi = ld(rb + 3, i)
                    cbyi = ld(rb + 4, i)
                    cbzi = ld(rb + 5, i)
                    xx = ld(rb + 0, i) - cbxi
                    xy = ld(rb + 1, i) - cbyi
                    xz = ld(rb + 2, i) - cbzi
                    yx = ld(rb + 3, j) - cbxi
                    yy = ld(rb + 4, j) - cbyi
                    yz = ld(rb + 5, j) - cbzi
                    nx2 = xx * xx + xy * xy + xz * xz
                    ny2 = yx * yx + yy * yy + yz * yz
                    dot = xx * yx + xy * yy + xz * yz
                    nx = _sqrtf(nx2)
                    ny = _sqrtf(ny2)
                    m1 = (nx > EPS) & (ny > EPS)
                    denom = jnp.where(m1, nx * ny, jnp.float32(1.0))
                    cth = dot / denom
                    good = m1 & ((jnp.float32(1.0) - cth * cth) > EPS) & vm
                    phi = _acosf(jnp.where(good, cth, jnp.float32(0.0)))

                    nlt = jnp.zeros((LN,), jnp.int32)
                    cutsel = cutv[0]
                    for k in range(ncut):
                        ck = cutv[k]
                        m = phi > ck
                        nlt = nlt + jnp.where(m, jnp.int32(1), jnp.int32(0))
                        if 1 <= k <= nseg - 1:
                            cutsel = jnp.where(m, ck, cutsel)
                    idx = jnp.clip(nlt - jnp.int32(1), jnp.int32(0),
                                   jnp.int32(nseg - 1))
                    o = jnp.where(good, p * jnp.int32(nseg) + idx, 0)

                    offs[pl.ds(t * LN, LN)] = lax.shift_right_logical(o, 1)
                    subs[pl.ds(t * LN, LN)] = (o & jnp.int32(1)) * jnp.int32(4)
                    xls[pl.ds(t * LN, LN)] = phi - cutsel
                    wts[pl.ds(t * LN, LN)] = jnp.where(good, jnp.float32(1.0),
                                                       jnp.float32(0.0))

                pltpu.async_copy(coeff_hbm.at[offs], rows, sem).wait()

                for t in range(SUB):
                    pidx = t * LN + iota
                    sub = subs[pl.ds(t * LN, LN)]
                    c0 = plsc.load_gather(rows, [pidx, sub])
                    c1 = plsc.load_gather(rows, [pidx, sub + jnp.int32(1)])
                    c2 = plsc.load_gather(rows, [pidx, sub + jnp.int32(2)])
                    c3 = plsc.load_gather(rows, [pidx, sub + jnp.int32(3)])
                    xl = xls[pl.ds(t * LN, LN)]
                    w = wts[pl.ds(t * LN, LN)]
                    tx = xl * xl
                    ret = (c3 + c2 * xl) + c1 * tx
                    ret = ret + c0 * (tx * xl)
                    plsc.addupdate(accv.at[pl.ds(0, LN)], w * ret)
                return carry

            return chunk_body

        for b in range(Bn):
            lax.fori_loop(jnp.int32(0), nch, make_chunk_body(b), jnp.int32(0))

        pltpu.sync_copy(accv, out_hbm.at[wid])

    return sc_call


def kernel(CA, CB, coeff, cutoffs, mask):
    Bn, L, _ = CA.shape
    nseg = coeff.shape[2]
    ncut = cutoffs.shape[0]
    geom = jnp.concatenate(
        [jnp.swapaxes(CA, 1, 2), jnp.swapaxes(CB, 1, 2)], axis=1)
    geom = geom.reshape(Bn * 6, L)
    mask_i = mask.reshape(-1).astype(jnp.int32)
    coeff_flat = coeff.reshape(L * L * nseg * 4 // 8, 8)
    cut_bro = jnp.broadcast_to(cutoffs[:, None], (ncut, LN))
    sc_call = _make_sc_call(Bn, L, nseg, ncut)
    out = sc_call(mask_i, geom, cut_bro, coeff_flat)
    return jnp.sum(out)
